# baseline (speedup 1.0000x reference)
import functools

import jax
import jax.numpy as jnp
from jax.experimental import pallas as pl
from jax.experimental.pallas import tpu as pltpu

_N = 8192
_BI = 1024
_BJ = 1024


def _chamfer_body(ax, ay, az, bx, by, bz, asq, bsq, out_ref, cmin_ref):
    ni = _N // _BI
    nj = _N // _BJ

    cmin_ref[...] = jnp.full((1, _N), jnp.inf, jnp.float32)

    def i_step(i, row_sum):
        axb = ax[pl.ds(i * _BI, _BI), :]
        ayb = ay[pl.ds(i * _BI, _BI), :]
        azb = az[pl.ds(i * _BI, _BI), :]
        asqb = asq[pl.ds(i * _BI, _BI), :]

        def j_step(j, rmin):
            g = (
                axb * bx[:, pl.ds(j * _BJ, _BJ)]
                + ayb * by[:, pl.ds(j * _BJ, _BJ)]
                + azb * bz[:, pl.ds(j * _BJ, _BJ)]
            )
            d2 = asqb + bsq[:, pl.ds(j * _BJ, _BJ)] - 2.0 * g
            d2 = jnp.maximum(d2, 0.0)
            cmin_ref[:, pl.ds(j * _BJ, _BJ)] = jnp.minimum(
                cmin_ref[:, pl.ds(j * _BJ, _BJ)], jnp.min(d2, axis=0, keepdims=True)
            )
            return jnp.minimum(rmin, jnp.min(d2, axis=1, keepdims=True))

        rmin = jax.lax.fori_loop(
            0, nj, j_step, jnp.full((_BI, 1), jnp.inf, jnp.float32)
        )
        return row_sum + jnp.sum(jnp.sqrt(rmin))

    row_sum = jax.lax.fori_loop(0, ni, i_step, jnp.float32(0.0))
    col_sum = jnp.sum(jnp.sqrt(cmin_ref[...]))
    out_ref[0, 0] = (row_sum + col_sum) / jnp.float32(_N)


def _rn_bf16(x):
    u = jax.lax.bitcast_convert_type(x, jnp.uint32)
    u = (u + jnp.uint32(0x7FFF) + ((u >> 16) & jnp.uint32(1))) & jnp.uint32(
        0xFFFF0000
    )
    return jax.lax.bitcast_convert_type(u, jnp.float32)


@jax.jit
def kernel(pc1, pc2):
    a = pc1.reshape(-1, 3)
    b = pc2.reshape(-1, 3)
    asq = jnp.sum(a * a, axis=1, keepdims=True)
    bsq = jnp.sum(b * b, axis=1, keepdims=True).T
    a16 = _rn_bf16(a)
    b16 = _rn_bf16(b)
    ax, ay, az = a16[:, 0:1], a16[:, 1:2], a16[:, 2:3]
    bx, by, bz = b16[:, 0:1].T, b16[:, 1:2].T, b16[:, 2:3].T
    out = pl.pallas_call(
        _chamfer_body,
        out_shape=jax.ShapeDtypeStruct((1, 1), jnp.float32),
        in_specs=[pl.BlockSpec(memory_space=pltpu.VMEM)] * 8,
        out_specs=pl.BlockSpec(memory_space=pltpu.SMEM),
        scratch_shapes=[pltpu.VMEM((1, _N), jnp.float32)],
    )(ax, ay, az, bx, by, bz, asq, bsq)
    return out[0, 0]


# speedup vs baseline: 1.0822x; 1.0822x over previous
import functools

import jax
import jax.numpy as jnp
from jax.experimental import pallas as pl
from jax.experimental.pallas import tpu as pltpu

_N = 8192
_BI = 512
_BJ = 4096


def _chamfer_body(ax, ay, az, bx, by, bz, asq, bsq, out_ref, cmin_ref):
    ni = _N // _BI
    nj = _N // _BJ

    cmin_ref[...] = jnp.full((1, _N), jnp.inf, jnp.float32)

    def i_step(i, row_sum):
        axb = ax[pl.ds(i * _BI, _BI), :]
        ayb = ay[pl.ds(i * _BI, _BI), :]
        azb = az[pl.ds(i * _BI, _BI), :]
        asqb = asq[pl.ds(i * _BI, _BI), :]

        def j_step(j, rmin):
            jsl = pl.ds(j * _BJ, _BJ)
            h = axb * bx[:, jsl] + ayb * by[:, jsl] + azb * bz[:, jsl] + bsq[:, jsl]
            cmin_ref[:, jsl] = jnp.minimum(
                cmin_ref[:, jsl], jnp.min(asqb + h, axis=0, keepdims=True)
            )
            return jnp.minimum(rmin, jnp.min(h, axis=1, keepdims=True))

        rmin = jax.lax.fori_loop(
            0, nj, j_step, jnp.full((_BI, 1), jnp.inf, jnp.float32)
        )
        rmin = jnp.maximum(rmin + asqb, 0.0)
        return row_sum + jnp.sum(jnp.sqrt(rmin))

    row_sum = jax.lax.fori_loop(0, ni, i_step, jnp.float32(0.0))
    col_sum = jnp.sum(jnp.sqrt(jnp.maximum(cmin_ref[...], 0.0)))
    out_ref[0, 0] = (row_sum + col_sum) / jnp.float32(_N)


def _rn_bf16(x):
    u = jax.lax.bitcast_convert_type(x, jnp.uint32)
    u = (u + jnp.uint32(0x7FFF) + ((u >> 16) & jnp.uint32(1))) & jnp.uint32(
        0xFFFF0000
    )
    return jax.lax.bitcast_convert_type(u, jnp.float32)


@jax.jit
def kernel(pc1, pc2):
    a = pc1.reshape(-1, 3)
    b = pc2.reshape(-1, 3)
    asq = jnp.sum(a * a, axis=1, keepdims=True)
    bsq = jnp.sum(b * b, axis=1, keepdims=True).T
    a16 = _rn_bf16(a) * jnp.float32(-2.0)
    b16 = _rn_bf16(b)
    ax, ay, az = a16[:, 0:1], a16[:, 1:2], a16[:, 2:3]
    bx, by, bz = b16[:, 0:1].T, b16[:, 1:2].T, b16[:, 2:3].T
    out = pl.pallas_call(
        _chamfer_body,
        out_shape=jax.ShapeDtypeStruct((1, 1), jnp.float32),
        in_specs=[pl.BlockSpec(memory_space=pltpu.VMEM)] * 8,
        out_specs=pl.BlockSpec(memory_space=pltpu.SMEM),
        scratch_shapes=[pltpu.VMEM((1, _N), jnp.float32)],
    )(ax, ay, az, bx, by, bz, asq, bsq)
    return out[0, 0]


# speedup vs baseline: 3.4506x; 3.1885x over previous
import functools

import jax
import jax.numpy as jnp
from jax.experimental import pallas as pl
from jax.experimental.pallas import tpu as pltpu

_N = 8192
_BI = 512


def _chamfer_body(a_ext, b_ext, out_ref, cmin_ref):
    ni = _N // _BI

    cmin_ref[...] = jnp.full((1, _N), jnp.inf, jnp.float32)

    def i_step(i, row_sum):
        f = jax.lax.dot_general(
            a_ext[pl.ds(i * _BI, _BI), :],
            b_ext[...],
            (((1,), (0,)), ((), ())),
            preferred_element_type=jnp.float32,
        )
        cmin_ref[...] = jnp.minimum(
            cmin_ref[...], jnp.min(f, axis=0, keepdims=True)
        )
        rmin = jnp.maximum(jnp.min(f, axis=1, keepdims=True), 0.0)
        return row_sum + jnp.sum(jnp.sqrt(rmin))

    row_sum = jax.lax.fori_loop(0, ni, i_step, jnp.float32(0.0))
    col_sum = jnp.sum(jnp.sqrt(jnp.maximum(cmin_ref[...], 0.0)))
    out_ref[0, 0] = (row_sum + col_sum) / jnp.float32(_N)


def _rn_bf16(x):
    u = jax.lax.bitcast_convert_type(x, jnp.uint32)
    u = (u + jnp.uint32(0x7FFF) + ((u >> 16) & jnp.uint32(1))) & jnp.uint32(
        0xFFFF0000
    )
    return jax.lax.bitcast_convert_type(u, jnp.float32)


@jax.jit
def kernel(pc1, pc2):
    a = pc1.reshape(-1, 3)
    b = pc2.reshape(-1, 3)
    asq = jnp.sum(a * a, axis=1, keepdims=True)
    bsq = jnp.sum(b * b, axis=1, keepdims=True)
    asq_hi = _rn_bf16(asq)
    asq_lo = asq - asq_hi
    bsq_hi = _rn_bf16(bsq)
    bsq_lo = bsq - bsq_hi
    a16 = _rn_bf16(a) * jnp.float32(-2.0)
    b16 = _rn_bf16(b)
    ones = jnp.ones_like(asq)
    zeros = jnp.zeros_like(asq)
    a_ext = jnp.concatenate(
        [a16, asq_hi, asq_lo, ones, ones, zeros], axis=1
    ).astype(jnp.bfloat16)
    b_ext = (
        jnp.concatenate([b16, ones, ones, bsq_hi, bsq_lo, zeros], axis=1)
        .astype(jnp.bfloat16)
        .T
    )
    out = pl.pallas_call(
        _chamfer_body,
        out_shape=jax.ShapeDtypeStruct((1, 1), jnp.float32),
        in_specs=[pl.BlockSpec(memory_space=pltpu.VMEM)] * 2,
        out_specs=pl.BlockSpec(memory_space=pltpu.SMEM),
        scratch_shapes=[pltpu.VMEM((1, _N), jnp.float32)],
    )(a_ext, b_ext)
    return out[0, 0]
